# baseline (speedup 1.0000x reference)
import functools

import jax
import jax.numpy as jnp
from jax import lax
from jax.experimental import pallas as pl
from jax.experimental.pallas import tpu as pltpu
from jax.experimental.pallas import tpu_sc as plsc

_B = 16
_F = 64
_N = 100000
_K = 4
_M = 16
_BN = 8192
_G = (_N + _BN - 1) // _BN

_INF = float("inf")
_IBIG = 2 ** 30


def _presel_body(x_ref, fp_ref, idx_ref, rd_ref, ri_ref):
    i = pl.program_id(0)

    @pl.when(i == 0)
    def _init():
        rd_ref[...] = jnp.full((_B, _M), _INF, jnp.float32)
        ri_ref[...] = jnp.zeros((_B, _M), jnp.int32)

    x = x_ref[...]
    xb = x.astype(jnp.bfloat16)
    xsq = jnp.sum(x * x, axis=1, keepdims=True)
    fb = fp_ref[...].astype(jnp.bfloat16)
    fb2 = fb * fb
    ones_row = jnp.ones((1, _F), jnp.bfloat16)
    fsq = jax.lax.dot_general(
        ones_row, fb2, (((1,), (1,)), ((), ())),
        preferred_element_type=jnp.float32)
    a = jax.lax.dot_general(
        xb, fb, (((1,), (1,)), ((), ())),
        preferred_element_type=jnp.float32)
    d2 = fsq - 2.0 * a + (xsq + 16.0)

    col = jax.lax.broadcasted_iota(jnp.int32, (_B, _BN), 1)
    d2 = jnp.where(col + i * _BN < _N, d2, _INF)

    bds, bis = [], []
    for _ in range(_M):
        m = jnp.min(d2, axis=1, keepdims=True)
        sel = jnp.min(jnp.where(d2 == m, col, _BN), axis=1, keepdims=True)
        bds.append(m)
        bis.append(sel + i * _BN)
        d2 = jnp.where(col == sel, _INF, d2)

    cd = jnp.concatenate([rd_ref[...]] + bds, axis=1)
    ci = jnp.concatenate([ri_ref[...]] + bis, axis=1)
    cid = jax.lax.broadcasted_iota(jnp.int32, (_B, 2 * _M), 1)
    nds, nis = [], []
    for _ in range(_M):
        m = jnp.min(cd, axis=1, keepdims=True)
        sel = jnp.min(jnp.where(cd == m, cid, 2 * _M), axis=1, keepdims=True)
        oh = cid == sel
        nis.append(jnp.sum(jnp.where(oh, ci, 0), axis=1, keepdims=True))
        nds.append(m)
        cd = jnp.where(oh, _INF, cd)
    rd_ref[...] = jnp.concatenate(nds, axis=1)
    ri_ref[...] = jnp.concatenate(nis, axis=1)

    @pl.when(i == _G - 1)
    def _finish():
        idx_ref[...] = ri_ref[...]


def _presel_call(x, fingerprints):
    return pl.pallas_call(
        _presel_body,
        grid=(_G,),
        in_specs=[
            pl.BlockSpec((_B, _F), lambda i: (0, 0)),
            pl.BlockSpec((_BN, _F), lambda i: (i, 0)),
        ],
        out_specs=pl.BlockSpec((_B, _M), lambda i: (0, 0)),
        out_shape=jax.ShapeDtypeStruct((_B, _M), jnp.int32),
        scratch_shapes=[
            pltpu.VMEM((_B, _M), jnp.float32),
            pltpu.VMEM((_B, _M), jnp.int32),
        ],
        compiler_params=pltpu.CompilerParams(
            dimension_semantics=("arbitrary",)),
    )(x, fingerprints)


def _inv_w_denom(v):
    y = v * (1.0 / 12.0) + 4.0
    for _ in range(3):
        y = 0.5 * (y + v / y)
    return v + 2e-6 * y + 1e-12


def _rescore_body(idxT_hbm, x_hbm, fp_hbm, posx_hbm, posy_hbm, out_hbm,
                  i_v, x_v, fpr_v, px4_v, py4_v, o_v, sem):
    cid = lax.axis_index("c")
    sid = lax.axis_index("s")

    @pl.when(jnp.logical_and(cid == 0, sid == 0))
    def _():
        pltpu.sync_copy(idxT_hbm, i_v)
        pltpu.sync_copy(x_hbm, x_v)
        copies = []
        for k in range(_M):
            row = i_v[k, :]
            for b in range(_B):
                copies.append(pltpu.async_copy(
                    fp_hbm.at[row[b]], fpr_v.at[k * _B + b], sem))
        for c in copies:
            c.wait()

        iota = lax.iota(jnp.int32, _B)
        d2s = [jnp.zeros((_B,), jnp.float32) for _ in range(_M)]
        for b in range(_B):
            x4 = [x_v[b, pl.ds(j * 16, 16)] for j in range(_F // 16)]
            for k in range(_M):
                c = k * _B + b
                acc = None
                for j in range(_F // 16):
                    t = x4[j] - fpr_v[c, pl.ds(j * 16, 16)]
                    acc = t * t if acc is None else acc + t * t
                for sh in (8, 4, 2, 1):
                    acc = acc + acc.at[lax.bitwise_xor(iota, sh)].get(
                        mode="promise_in_bounds")
                d2s[k] = jnp.where(iota == b, acc, d2s[k])

        gis = [i_v[k, :] for k in range(_M)]

        pos_copies = []
        ws = []
        for r in range(_K):
            m = d2s[0]
            for k in range(1, _M):
                m = jnp.minimum(m, d2s[k])
            gsel = jnp.full((_B,), _IBIG, jnp.int32)
            for k in range(_M):
                gsel = jnp.minimum(gsel, jnp.where(d2s[k] == m, gis[k], _IBIG))
            for k in range(_M):
                d2s[k] = jnp.where(gis[k] == gsel, _INF, d2s[k])
            pos_copies.append(
                (pltpu.async_copy(posx_hbm.at[gsel], px4_v.at[r], sem),
                 pltpu.async_copy(posy_hbm.at[gsel], py4_v.at[r], sem)))
            w = 1.0 / _inv_w_denom(m + 1e-12)
            ws.append(w)
        for cx, cy in pos_copies:
            cx.wait()
            cy.wait()
        wsum = ws[0] + ws[1] + ws[2] + ws[3] + 1e-12
        accx = ws[0] * px4_v[0, :]
        accy = ws[0] * py4_v[0, :]
        for r in range(1, _K):
            accx = accx + ws[r] * px4_v[r, :]
            accy = accy + ws[r] * py4_v[r, :]
        o_v[0, :] = accx / wsum
        o_v[1, :] = accy / wsum
        pltpu.sync_copy(o_v, out_hbm)


@functools.cache
def _build_rescore():
    return functools.partial(
        pl.kernel,
        out_type=jax.ShapeDtypeStruct((2, _B), jnp.float32),
        mesh=plsc.VectorSubcoreMesh(core_axis_name="c", subcore_axis_name="s"),
        scratch_types=[
            pltpu.VMEM((_M, _B), jnp.int32),
            pltpu.VMEM((_B, _F), jnp.float32),
            pltpu.VMEM((_M * _B, _F), jnp.float32),
            pltpu.VMEM((_K, _B), jnp.float32),
            pltpu.VMEM((_K, _B), jnp.float32),
            pltpu.VMEM((2, _B), jnp.float32),
            pltpu.SemaphoreType.DMA,
        ],
    )(_rescore_body)


@jax.jit
def kernel(x, fingerprints, positions):
    idx = _presel_call(x, fingerprints)
    idxT = idx.T
    posx = positions[:, 0]
    posy = positions[:, 1]
    out = _build_rescore()(idxT, x, fingerprints, posx, posy)
    return out.T


# speedup vs baseline: 1.0496x; 1.0496x over previous
import functools

import jax
import jax.numpy as jnp
from jax import lax
from jax.experimental import pallas as pl
from jax.experimental.pallas import tpu as pltpu
from jax.experimental.pallas import tpu_sc as plsc

_B = 16
_F = 64
_N = 100000
_K = 4
_M = 16
_BN = 8192
_G = (_N + _BN - 1) // _BN

_INF = float("inf")
_IBIG = 2 ** 30
_IMAX = 2 ** 31 - 1


def _presel_body(x_ref, fp_ref, idx_ref, rd_ref, ri_ref):
    i = pl.program_id(0)

    @pl.when(i == 0)
    def _init():
        rd_ref[...] = jnp.full((_B, _M), _IMAX, jnp.int32)
        ri_ref[...] = jnp.zeros((_B, _M), jnp.int32)

    x = x_ref[...]
    xb = x.astype(jnp.bfloat16)
    xsq = jnp.sum(x * x, axis=1, keepdims=True)
    fb = fp_ref[...].astype(jnp.bfloat16)
    fb2 = fb * fb
    ones_row = jnp.ones((1, _F), jnp.bfloat16)
    fsq = jax.lax.dot_general(
        ones_row, fb2, (((1,), (1,)), ((), ())),
        preferred_element_type=jnp.float32)
    a = jax.lax.dot_general(
        xb, fb, (((1,), (1,)), ((), ())),
        preferred_element_type=jnp.float32)
    d2 = fsq - 2.0 * a + (xsq + 32.0)

    col = jax.lax.broadcasted_iota(jnp.int32, (_B, _BN), 1)
    d2 = jnp.where(col + i * _BN < _N, d2, _INF)

    bits = jax.lax.bitcast_convert_type(d2, jnp.int32)
    keys = jnp.bitwise_or(jnp.bitwise_and(bits, ~0x1FFF), col)

    bks = []
    for _ in range(_M):
        m = jnp.min(keys, axis=1, keepdims=True)
        bks.append(m)
        keys = jnp.where(keys == m, _IMAX, keys)

    bvs = [jnp.bitwise_and(k, ~0x1FFF) for k in bks]
    bis = [jnp.bitwise_and(k, 0x1FFF) + i * _BN for k in bks]
    cd = jnp.concatenate([rd_ref[...]] + bvs, axis=1)
    ci = jnp.concatenate([ri_ref[...]] + bis, axis=1)
    cid = jax.lax.broadcasted_iota(jnp.int32, (_B, 2 * _M), 1)
    nds, nis = [], []
    for _ in range(_M):
        m = jnp.min(cd, axis=1, keepdims=True)
        sel = jnp.min(jnp.where(cd == m, cid, 2 * _M), axis=1, keepdims=True)
        oh = cid == sel
        nis.append(jnp.sum(jnp.where(oh, ci, 0), axis=1, keepdims=True))
        nds.append(m)
        cd = jnp.where(oh, _IMAX, cd)
    rd_ref[...] = jnp.concatenate(nds, axis=1)
    ri_ref[...] = jnp.concatenate(nis, axis=1)

    @pl.when(i == _G - 1)
    def _finish():
        idx_ref[...] = ri_ref[...]


def _presel_call(x, fingerprints):
    return pl.pallas_call(
        _presel_body,
        grid=(_G,),
        in_specs=[
            pl.BlockSpec((_B, _F), lambda i: (0, 0)),
            pl.BlockSpec((_BN, _F), lambda i: (i, 0)),
        ],
        out_specs=pl.BlockSpec((_B, _M), lambda i: (0, 0)),
        out_shape=jax.ShapeDtypeStruct((_B, _M), jnp.int32),
        scratch_shapes=[
            pltpu.VMEM((_B, _M), jnp.int32),
            pltpu.VMEM((_B, _M), jnp.int32),
        ],
        compiler_params=pltpu.CompilerParams(
            dimension_semantics=("arbitrary",)),
    )(x, fingerprints)


def _inv_w_denom(v):
    y = v * (1.0 / 12.0) + 4.0
    for _ in range(3):
        y = 0.5 * (y + v / y)
    return v + 2e-6 * y + 1e-12


def _rescore_body(idxT_hbm, x_hbm, fp_hbm, posx_hbm, posy_hbm, out_hbm,
                  i_v, x_v, fpr_v, px4_v, py4_v, o_v, sem):
    cid = lax.axis_index("c")
    sid = lax.axis_index("s")

    @pl.when(jnp.logical_and(cid == 0, sid == 0))
    def _():
        pltpu.sync_copy(idxT_hbm, i_v)
        pltpu.sync_copy(x_hbm, x_v)
        copies = []
        for k in range(_M):
            row = i_v[k, :]
            for b in range(_B):
                copies.append(pltpu.async_copy(
                    fp_hbm.at[row[b]], fpr_v.at[k * _B + b], sem))
        for c in copies:
            c.wait()

        iota = lax.iota(jnp.int32, _B)
        d2s = [jnp.zeros((_B,), jnp.float32) for _ in range(_M)]
        for b in range(_B):
            x4 = [x_v[b, pl.ds(j * 16, 16)] for j in range(_F // 16)]
            for k in range(_M):
                c = k * _B + b
                acc = None
                for j in range(_F // 16):
                    t = x4[j] - fpr_v[c, pl.ds(j * 16, 16)]
                    acc = t * t if acc is None else acc + t * t
                for sh in (8, 4, 2, 1):
                    acc = acc + acc.at[lax.bitwise_xor(iota, sh)].get(
                        mode="promise_in_bounds")
                d2s[k] = jnp.where(iota == b, acc, d2s[k])

        gis = [i_v[k, :] for k in range(_M)]

        pos_copies = []
        ws = []
        for r in range(_K):
            m = d2s[0]
            for k in range(1, _M):
                m = jnp.minimum(m, d2s[k])
            gsel = jnp.full((_B,), _IBIG, jnp.int32)
            for k in range(_M):
                gsel = jnp.minimum(gsel, jnp.where(d2s[k] == m, gis[k], _IBIG))
            for k in range(_M):
                d2s[k] = jnp.where(gis[k] == gsel, _INF, d2s[k])
            pos_copies.append(
                (pltpu.async_copy(posx_hbm.at[gsel], px4_v.at[r], sem),
                 pltpu.async_copy(posy_hbm.at[gsel], py4_v.at[r], sem)))
            w = 1.0 / _inv_w_denom(m + 1e-12)
            ws.append(w)
        for cx, cy in pos_copies:
            cx.wait()
            cy.wait()
        wsum = ws[0] + ws[1] + ws[2] + ws[3] + 1e-12
        accx = ws[0] * px4_v[0, :]
        accy = ws[0] * py4_v[0, :]
        for r in range(1, _K):
            accx = accx + ws[r] * px4_v[r, :]
            accy = accy + ws[r] * py4_v[r, :]
        o_v[0, :] = accx / wsum
        o_v[1, :] = accy / wsum
        pltpu.sync_copy(o_v, out_hbm)


@functools.cache
def _build_rescore():
    return functools.partial(
        pl.kernel,
        out_type=jax.ShapeDtypeStruct((2, _B), jnp.float32),
        mesh=plsc.VectorSubcoreMesh(core_axis_name="c", subcore_axis_name="s"),
        scratch_types=[
            pltpu.VMEM((_M, _B), jnp.int32),
            pltpu.VMEM((_B, _F), jnp.float32),
            pltpu.VMEM((_M * _B, _F), jnp.float32),
            pltpu.VMEM((_K, _B), jnp.float32),
            pltpu.VMEM((_K, _B), jnp.float32),
            pltpu.VMEM((2, _B), jnp.float32),
            pltpu.SemaphoreType.DMA,
        ],
    )(_rescore_body)


@jax.jit
def kernel(x, fingerprints, positions):
    idx = _presel_call(x, fingerprints)
    idxT = idx.T
    posx = positions[:, 0]
    posy = positions[:, 1]
    out = _build_rescore()(idxT, x, fingerprints, posx, posy)
    return out.T


# speedup vs baseline: 2.0641x; 1.9665x over previous
import functools

import jax
import jax.numpy as jnp
from jax import lax
from jax.experimental import pallas as pl
from jax.experimental.pallas import tpu as pltpu
from jax.experimental.pallas import tpu_sc as plsc

_B = 16
_F = 64
_N = 100000
_K = 4
_M = 16
_BN = 8192
_G = (_N + _BN - 1) // _BN

_INF = float("inf")
_IBIG = 2 ** 30
_IMAX = 2 ** 31 - 1
_NC = 256


def _cmpx(ka, ga, kb, gb):
    le = ka <= kb
    return (jnp.minimum(ka, kb), jnp.where(le, ga, gb),
            jnp.maximum(ka, kb), jnp.where(le, gb, ga))


def _presel_body(x_ref, fp_ref, idx_ref, rk_ref, rg_ref):
    i = pl.program_id(0)

    @pl.when(i == 0)
    def _init():
        rk_ref[...] = jnp.full((3, _B, _NC), _IMAX, jnp.int32)
        rg_ref[...] = jnp.zeros((3, _B, _NC), jnp.int32)

    x = x_ref[...]
    xb = x.astype(jnp.bfloat16)
    xsq = jnp.sum(x * x, axis=1, keepdims=True)
    fb = fp_ref[...].astype(jnp.bfloat16)
    fb2 = fb * fb
    ones_row = jnp.ones((1, _F), jnp.bfloat16)
    fsq = jax.lax.dot_general(
        ones_row, fb2, (((1,), (1,)), ((), ())),
        preferred_element_type=jnp.float32)
    a = jax.lax.dot_general(
        xb, fb, (((1,), (1,)), ((), ())),
        preferred_element_type=jnp.float32)
    d2 = fsq - 2.0 * a + (xsq + 32.0)

    col = jax.lax.broadcasted_iota(jnp.int32, (_B, _BN), 1)
    d2 = jnp.where(col + i * _BN < _N, d2, _INF)

    bits = jax.lax.bitcast_convert_type(d2, jnp.int32)
    keys = jnp.bitwise_or(jnp.bitwise_and(bits, ~0x1FFF), col)

    groups = [keys[:, o * _NC:(o + 1) * _NC] for o in range(_BN // _NC)]
    bw = []
    for _ in range(3):
        t = list(groups)
        while len(t) > 1:
            t = [jnp.minimum(t[j], t[j + 1]) for j in range(0, len(t), 2)]
        w = t[0]
        bw.append(w)
        groups = [jnp.where(g == w, _IMAX, g) for g in groups]
    bg = [jnp.bitwise_and(w, 0x1FFF) + i * _BN for w in bw]

    a1, a2, a3 = rk_ref[0], rk_ref[1], rk_ref[2]
    e1, e2, e3 = rg_ref[0], rg_ref[1], rg_ref[2]
    b1, b2, b3 = bw
    f1, f2, f3 = bg
    m1k, m1g, x1k, x1g = _cmpx(a1, e1, b1, f1)
    m2k, m2g, x2k, x2g = _cmpx(a2, e2, b2, f2)
    m3k, m3g, _, _ = _cmpx(a3, e3, b3, f3)
    r2k, r2g, t1k, t1g = _cmpx(x1k, x1g, m2k, m2g)
    t2k, t2g, _, _ = _cmpx(x2k, x2g, m3k, m3g)
    r3k, r3g, _, _ = _cmpx(t1k, t1g, t2k, t2g)
    rk_ref[0], rg_ref[0] = m1k, m1g
    rk_ref[1], rg_ref[1] = r2k, r2g
    rk_ref[2], rg_ref[2] = r3k, r3g

    @pl.when(i == _G - 1)
    def _finish():
        ck = jnp.concatenate([rk_ref[0], rk_ref[1], rk_ref[2]], axis=1)
        cg = jnp.concatenate([rg_ref[0], rg_ref[1], rg_ref[2]], axis=1)
        cid = jax.lax.broadcasted_iota(jnp.int32, (_B, 3 * _NC), 1)
        for r in range(_M):
            m = jnp.min(ck, axis=1, keepdims=True)
            sel = jnp.min(jnp.where(ck == m, cid, 3 * _NC),
                          axis=1, keepdims=True)
            oh = cid == sel
            idx_ref[:, r:r + 1] = jnp.sum(jnp.where(oh, cg, 0),
                                          axis=1, keepdims=True)
            ck = jnp.where(oh, _IMAX, ck)


def _presel_call(x, fingerprints):
    return pl.pallas_call(
        _presel_body,
        grid=(_G,),
        in_specs=[
            pl.BlockSpec((_B, _F), lambda i: (0, 0)),
            pl.BlockSpec((_BN, _F), lambda i: (i, 0)),
        ],
        out_specs=pl.BlockSpec((_B, _M), lambda i: (0, 0)),
        out_shape=jax.ShapeDtypeStruct((_B, _M), jnp.int32),
        scratch_shapes=[
            pltpu.VMEM((3, _B, _NC), jnp.int32),
            pltpu.VMEM((3, _B, _NC), jnp.int32),
        ],
        compiler_params=pltpu.CompilerParams(
            dimension_semantics=("arbitrary",)),
    )(x, fingerprints)


def _inv_w_denom(v):
    y = v * (1.0 / 12.0) + 4.0
    for _ in range(3):
        y = 0.5 * (y + v / y)
    return v + 2e-6 * y + 1e-12


def _rescore_body(idxT_hbm, x_hbm, fp_hbm, posx_hbm, posy_hbm, out_hbm,
                  i_v, x_v, fpr_v, px4_v, py4_v, o_v, sem):
    cid = lax.axis_index("c")
    sid = lax.axis_index("s")

    @pl.when(jnp.logical_and(cid == 0, sid == 0))
    def _():
        pltpu.sync_copy(idxT_hbm, i_v)
        pltpu.sync_copy(x_hbm, x_v)
        copies = []
        for k in range(_M):
            row = i_v[k, :]
            for b in range(_B):
                copies.append(pltpu.async_copy(
                    fp_hbm.at[row[b]], fpr_v.at[k * _B + b], sem))
        for c in copies:
            c.wait()

        iota = lax.iota(jnp.int32, _B)
        d2s = [jnp.zeros((_B,), jnp.float32) for _ in range(_M)]
        for b in range(_B):
            x4 = [x_v[b, pl.ds(j * 16, 16)] for j in range(_F // 16)]
            for k in range(_M):
                c = k * _B + b
                acc = None
                for j in range(_F // 16):
                    t = x4[j] - fpr_v[c, pl.ds(j * 16, 16)]
                    acc = t * t if acc is None else acc + t * t
                for sh in (8, 4, 2, 1):
                    acc = acc + acc.at[lax.bitwise_xor(iota, sh)].get(
                        mode="promise_in_bounds")
                d2s[k] = jnp.where(iota == b, acc, d2s[k])

        gis = [i_v[k, :] for k in range(_M)]

        pos_copies = []
        ws = []
        for r in range(_K):
            m = d2s[0]
            for k in range(1, _M):
                m = jnp.minimum(m, d2s[k])
            gsel = jnp.full((_B,), _IBIG, jnp.int32)
            for k in range(_M):
                gsel = jnp.minimum(gsel, jnp.where(d2s[k] == m, gis[k], _IBIG))
            for k in range(_M):
                d2s[k] = jnp.where(gis[k] == gsel, _INF, d2s[k])
            pos_copies.append(
                (pltpu.async_copy(posx_hbm.at[gsel], px4_v.at[r], sem),
                 pltpu.async_copy(posy_hbm.at[gsel], py4_v.at[r], sem)))
            w = 1.0 / _inv_w_denom(m + 1e-12)
            ws.append(w)
        for cx, cy in pos_copies:
            cx.wait()
            cy.wait()
        wsum = ws[0] + ws[1] + ws[2] + ws[3] + 1e-12
        accx = ws[0] * px4_v[0, :]
        accy = ws[0] * py4_v[0, :]
        for r in range(1, _K):
            accx = accx + ws[r] * px4_v[r, :]
            accy = accy + ws[r] * py4_v[r, :]
        o_v[0, :] = accx / wsum
        o_v[1, :] = accy / wsum
        pltpu.sync_copy(o_v, out_hbm)


@functools.cache
def _build_rescore():
    return functools.partial(
        pl.kernel,
        out_type=jax.ShapeDtypeStruct((2, _B), jnp.float32),
        mesh=plsc.VectorSubcoreMesh(core_axis_name="c", subcore_axis_name="s"),
        scratch_types=[
            pltpu.VMEM((_M, _B), jnp.int32),
            pltpu.VMEM((_B, _F), jnp.float32),
            pltpu.VMEM((_M * _B, _F), jnp.float32),
            pltpu.VMEM((_K, _B), jnp.float32),
            pltpu.VMEM((_K, _B), jnp.float32),
            pltpu.VMEM((2, _B), jnp.float32),
            pltpu.SemaphoreType.DMA,
        ],
    )(_rescore_body)


@jax.jit
def kernel(x, fingerprints, positions):
    idx = _presel_call(x, fingerprints)
    idxT = idx.T
    posx = positions[:, 0]
    posy = positions[:, 1]
    out = _build_rescore()(idxT, x, fingerprints, posx, posy)
    return out.T


# speedup vs baseline: 2.1012x; 1.0179x over previous
import functools

import jax
import jax.numpy as jnp
from jax import lax
from jax.experimental import pallas as pl
from jax.experimental.pallas import tpu as pltpu
from jax.experimental.pallas import tpu_sc as plsc

_B = 16
_F = 64
_N = 100000
_K = 4
_M = 16
_BN = 16384
_G = (_N + _BN - 1) // _BN
_CMASK = _BN - 1

_INF = float("inf")
_IBIG = 2 ** 30
_IMAX = 2 ** 31 - 1
_NC = 256


def _cmpx(ka, ga, kb, gb):
    le = ka <= kb
    return (jnp.minimum(ka, kb), jnp.where(le, ga, gb),
            jnp.maximum(ka, kb), jnp.where(le, gb, ga))


def _presel_body(x_ref, fp_ref, idx_ref, rk_ref, rg_ref):
    i = pl.program_id(0)

    @pl.when(i == 0)
    def _init():
        rk_ref[...] = jnp.full((3, _B, _NC), _IMAX, jnp.int32)
        rg_ref[...] = jnp.zeros((3, _B, _NC), jnp.int32)

    x = x_ref[...]
    xb = x.astype(jnp.bfloat16)
    xsq = jnp.sum(x * x, axis=1, keepdims=True)
    fb = fp_ref[...].astype(jnp.bfloat16)
    fb2 = fb * fb
    ones_row = jnp.ones((1, _F), jnp.bfloat16)
    fsq = jax.lax.dot_general(
        ones_row, fb2, (((1,), (1,)), ((), ())),
        preferred_element_type=jnp.float32)
    a = jax.lax.dot_general(
        xb, fb, (((1,), (1,)), ((), ())),
        preferred_element_type=jnp.float32)
    d2 = fsq - 2.0 * a + (xsq + 32.0)

    col = jax.lax.broadcasted_iota(jnp.int32, (_B, _BN), 1)
    d2 = jnp.where(col + i * _BN < _N, d2, _INF)

    bits = jax.lax.bitcast_convert_type(d2, jnp.int32)
    keys = jnp.bitwise_or(jnp.bitwise_and(bits, ~_CMASK), col)

    groups = [keys[:, o * _NC:(o + 1) * _NC] for o in range(_BN // _NC)]
    bw = []
    for _ in range(3):
        t = list(groups)
        while len(t) > 1:
            t = [jnp.minimum(t[j], t[j + 1]) for j in range(0, len(t), 2)]
        w = t[0]
        bw.append(w)
        groups = [jnp.where(g == w, _IMAX, g) for g in groups]
    bg = [jnp.bitwise_and(w, _CMASK) + i * _BN for w in bw]

    a1, a2, a3 = rk_ref[0], rk_ref[1], rk_ref[2]
    e1, e2, e3 = rg_ref[0], rg_ref[1], rg_ref[2]
    b1, b2, b3 = bw
    f1, f2, f3 = bg
    m1k, m1g, x1k, x1g = _cmpx(a1, e1, b1, f1)
    m2k, m2g, x2k, x2g = _cmpx(a2, e2, b2, f2)
    m3k, m3g, _, _ = _cmpx(a3, e3, b3, f3)
    r2k, r2g, t1k, t1g = _cmpx(x1k, x1g, m2k, m2g)
    t2k, t2g, _, _ = _cmpx(x2k, x2g, m3k, m3g)
    r3k, r3g, _, _ = _cmpx(t1k, t1g, t2k, t2g)
    rk_ref[0], rg_ref[0] = m1k, m1g
    rk_ref[1], rg_ref[1] = r2k, r2g
    rk_ref[2], rg_ref[2] = r3k, r3g

    @pl.when(i == _G - 1)
    def _finish():
        ck = jnp.concatenate([rk_ref[0], rk_ref[1], rk_ref[2]], axis=1)
        cg = jnp.concatenate([rg_ref[0], rg_ref[1], rg_ref[2]], axis=1)
        cid = jax.lax.broadcasted_iota(jnp.int32, (_B, 3 * _NC), 1)
        for r in range(_M):
            m = jnp.min(ck, axis=1, keepdims=True)
            sel = jnp.min(jnp.where(ck == m, cid, 3 * _NC),
                          axis=1, keepdims=True)
            oh = cid == sel
            idx_ref[:, r:r + 1] = jnp.sum(jnp.where(oh, cg, 0),
                                          axis=1, keepdims=True)
            ck = jnp.where(oh, _IMAX, ck)


def _presel_call(x, fingerprints):
    return pl.pallas_call(
        _presel_body,
        grid=(_G,),
        in_specs=[
            pl.BlockSpec((_B, _F), lambda i: (0, 0)),
            pl.BlockSpec((_BN, _F), lambda i: (i, 0)),
        ],
        out_specs=pl.BlockSpec((_B, _M), lambda i: (0, 0)),
        out_shape=jax.ShapeDtypeStruct((_B, _M), jnp.int32),
        scratch_shapes=[
            pltpu.VMEM((3, _B, _NC), jnp.int32),
            pltpu.VMEM((3, _B, _NC), jnp.int32),
        ],
        compiler_params=pltpu.CompilerParams(
            dimension_semantics=("arbitrary",)),
    )(x, fingerprints)


def _inv_w_denom(v):
    y = v * (1.0 / 12.0) + 4.0
    for _ in range(3):
        y = 0.5 * (y + v / y)
    return v + 2e-6 * y + 1e-12


def _rescore_body(idxT_hbm, x_hbm, fp_hbm, posx_hbm, posy_hbm, out_hbm,
                  i_v, x_v, fpr_v, px4_v, py4_v, o_v, sem):
    cid = lax.axis_index("c")
    sid = lax.axis_index("s")

    @pl.when(jnp.logical_and(cid == 0, sid == 0))
    def _():
        pltpu.sync_copy(idxT_hbm, i_v)
        pltpu.sync_copy(x_hbm, x_v)
        copies = []
        for k in range(_M):
            row = i_v[k, :]
            for b in range(_B):
                copies.append(pltpu.async_copy(
                    fp_hbm.at[row[b]], fpr_v.at[k * _B + b], sem))
        for c in copies:
            c.wait()

        iota = lax.iota(jnp.int32, _B)
        d2s = [jnp.zeros((_B,), jnp.float32) for _ in range(_M)]
        for b in range(_B):
            x4 = [x_v[b, pl.ds(j * 16, 16)] for j in range(_F // 16)]
            for k in range(_M):
                c = k * _B + b
                acc = None
                for j in range(_F // 16):
                    t = x4[j] - fpr_v[c, pl.ds(j * 16, 16)]
                    acc = t * t if acc is None else acc + t * t
                for sh in (8, 4, 2, 1):
                    acc = acc + acc.at[lax.bitwise_xor(iota, sh)].get(
                        mode="promise_in_bounds")
                d2s[k] = jnp.where(iota == b, acc, d2s[k])

        gis = [i_v[k, :] for k in range(_M)]

        pos_copies = []
        ws = []
        for r in range(_K):
            m = d2s[0]
            for k in range(1, _M):
                m = jnp.minimum(m, d2s[k])
            gsel = jnp.full((_B,), _IBIG, jnp.int32)
            for k in range(_M):
                gsel = jnp.minimum(gsel, jnp.where(d2s[k] == m, gis[k], _IBIG))
            for k in range(_M):
                d2s[k] = jnp.where(gis[k] == gsel, _INF, d2s[k])
            pos_copies.append(
                (pltpu.async_copy(posx_hbm.at[gsel], px4_v.at[r], sem),
                 pltpu.async_copy(posy_hbm.at[gsel], py4_v.at[r], sem)))
            w = 1.0 / _inv_w_denom(m + 1e-12)
            ws.append(w)
        for cx, cy in pos_copies:
            cx.wait()
            cy.wait()
        wsum = ws[0] + ws[1] + ws[2] + ws[3] + 1e-12
        accx = ws[0] * px4_v[0, :]
        accy = ws[0] * py4_v[0, :]
        for r in range(1, _K):
            accx = accx + ws[r] * px4_v[r, :]
            accy = accy + ws[r] * py4_v[r, :]
        o_v[0, :] = accx / wsum
        o_v[1, :] = accy / wsum
        pltpu.sync_copy(o_v, out_hbm)


@functools.cache
def _build_rescore():
    return functools.partial(
        pl.kernel,
        out_type=jax.ShapeDtypeStruct((2, _B), jnp.float32),
        mesh=plsc.VectorSubcoreMesh(core_axis_name="c", subcore_axis_name="s"),
        scratch_types=[
            pltpu.VMEM((_M, _B), jnp.int32),
            pltpu.VMEM((_B, _F), jnp.float32),
            pltpu.VMEM((_M * _B, _F), jnp.float32),
            pltpu.VMEM((_K, _B), jnp.float32),
            pltpu.VMEM((_K, _B), jnp.float32),
            pltpu.VMEM((2, _B), jnp.float32),
            pltpu.SemaphoreType.DMA,
        ],
    )(_rescore_body)


@jax.jit
def kernel(x, fingerprints, positions):
    idx = _presel_call(x, fingerprints)
    idxT = idx.T
    posx = positions[:, 0]
    posy = positions[:, 1]
    out = _build_rescore()(idxT, x, fingerprints, posx, posy)
    return out.T
